# baseline (speedup 1.0000x reference)
import functools

import jax
import jax.numpy as jnp
from jax import lax
from jax.experimental import pallas as pl
from jax.experimental.pallas import tpu as pltpu
from jax.experimental.pallas import tpu_sc as plsc

N_CODES = 8192
N_TOK = 8192
D = 256
C_COST = 0.25

BM = 256
BN = 512
BM3 = 512


def _argmin_body(flat_ref, wt_ref, sf_ref, idx_ref):
    f = flat_ref[...]
    sf = sf_ref[...]

    def chunk(t, carry):
        best_v, best_i = carry
        w = wt_ref[:, pl.ds(t * BN, BN)]
        mm = lax.dot_general(f, w, (((1,), (0,)), ((), ())),
                             preferred_element_type=jnp.float32)
        sw = jnp.sum(w * w, axis=0, keepdims=True)
        d = (sf + sw) - 2.0 * mm
        vmin = jnp.min(d, axis=1, keepdims=True)
        cols = t * BN + lax.broadcasted_iota(jnp.int32, (BM, BN), 1)
        imin = jnp.min(jnp.where(d == vmin, cols, jnp.int32(2**31 - 1)),
                       axis=1, keepdims=True)
        upd = vmin < best_v
        return (jnp.where(upd, vmin, best_v),
                jnp.where(upd, imin, best_i))

    v0 = jnp.full((BM, 1), jnp.inf, jnp.float32)
    i0 = jnp.zeros((BM, 1), jnp.int32)
    _, best_i = lax.fori_loop(0, N_CODES // BN, chunk, (v0, i0))
    idx_ref[...] = best_i


_argmin_call = pl.pallas_call(
    _argmin_body,
    grid=(N_TOK // BM,),
    in_specs=[
        pl.BlockSpec((BM, D), lambda i: (i, 0)),
        pl.BlockSpec((D, N_CODES), lambda i: (0, 0)),
        pl.BlockSpec((BM, 1), lambda i: (i, 0)),
    ],
    out_specs=pl.BlockSpec((BM, 1), lambda i: (i, 0)),
    out_shape=jax.ShapeDtypeStruct((N_TOK, 1), jnp.int32),
)


@functools.cache
def _make_sc_gather():
    info = plsc.get_sparse_core_info()
    nw = info.num_cores * info.num_subcores
    bpw = N_TOK // nw
    mesh = plsc.VectorSubcoreMesh(core_axis_name="c", subcore_axis_name="s")

    @functools.partial(
        pl.kernel, mesh=mesh,
        out_type=jax.ShapeDtypeStruct((N_TOK, D), jnp.float32),
        scratch_types=[
            pltpu.VMEM((bpw,), jnp.int32),
            pltpu.VMEM((bpw, D), jnp.float32),
            pltpu.SemaphoreType.DMA,
        ],
    )
    def gather(table_hbm, idx_hbm, out_hbm, idx_v, rows_v, sem):
        wid = lax.axis_index("s") * info.num_cores + lax.axis_index("c")
        base = wid * bpw
        pltpu.sync_copy(idx_hbm.at[pl.ds(base, bpw)], idx_v)
        pltpu.async_copy(table_hbm.at[idx_v], rows_v, sem).wait()
        pltpu.sync_copy(rows_v, out_hbm.at[pl.ds(base, bpw)])

    return gather


def _st_body(flat_ref, q_ref, st_ref, loss_ref, acc_ref):
    i = pl.program_id(0)
    x = flat_ref[...]
    q = q_ref[...]
    dif = q - x
    st_ref[...] = x + dif
    part = jnp.sum(dif * dif)

    @pl.when(i == 0)
    def _():
        acc_ref[0] = part

    @pl.when(i > 0)
    def _():
        acc_ref[0] = acc_ref[0] + part

    @pl.when(i == pl.num_programs(0) - 1)
    def _():
        loss_ref[0, 0] = acc_ref[0] * ((1.0 + C_COST) / (N_TOK * D))


_st_call = pl.pallas_call(
    _st_body,
    grid=(N_TOK // BM3,),
    in_specs=[
        pl.BlockSpec((BM3, D), lambda i: (i, 0)),
        pl.BlockSpec((BM3, D), lambda i: (i, 0)),
    ],
    out_specs=[
        pl.BlockSpec((BM3, D), lambda i: (i, 0)),
        pl.BlockSpec(memory_space=pltpu.SMEM),
    ],
    out_shape=[
        jax.ShapeDtypeStruct((N_TOK, D), jnp.float32),
        jax.ShapeDtypeStruct((1, 1), jnp.float32),
    ],
    scratch_shapes=[pltpu.SMEM((1,), jnp.float32)],
)


def kernel(inputs, W):
    x = jnp.transpose(inputs, (0, 2, 3, 4, 1))
    flat = x.reshape(-1, D)
    sf = jnp.sum(flat ** 2, axis=1, keepdims=True)
    idx = _argmin_call(flat, W.T, sf)
    q = _make_sc_gather()(W, idx.reshape(-1))
    st, loss = _st_call(flat, q)
    out = jnp.transpose(st.reshape(2, 16, 16, 16, D), (0, 4, 1, 2, 3))
    return (loss[0, 0], out)


# speedup vs baseline: 1.8941x; 1.8941x over previous
import functools

import jax
import jax.numpy as jnp
from jax import lax
from jax.experimental import pallas as pl
from jax.experimental.pallas import tpu as pltpu
from jax.experimental.pallas import tpu_sc as plsc

N_CODES = 8192
N_TOK = 8192
D = 256
C_COST = 0.25

BM = 256
BN = 1024
BM3 = 512


def _argmin_body(f2_ref, wt_ref, sf_ref, sw_ref, idx_ref):
    f2 = f2_ref[...]
    sf = sf_ref[...]
    cols = lax.broadcasted_iota(jnp.int32, (BM, BN), 1).astype(jnp.float32)

    def chunk(t, carry):
        best_v, best_i = carry
        w = wt_ref[:, pl.ds(t * BN, BN)]
        mm2 = lax.dot_general(f2, w, (((1,), (0,)), ((), ())),
                              preferred_element_type=jnp.float32)
        sw = sw_ref[:, pl.ds(t * BN, BN)]
        d = (sf + sw) - mm2
        vmin = jnp.min(d, axis=1, keepdims=True)
        imin = jnp.min(jnp.where(d == vmin, cols, jnp.float32(jnp.inf)),
                       axis=1, keepdims=True)
        imin = imin + jnp.float32(BN) * t.astype(jnp.float32)
        upd = vmin < best_v
        return (jnp.where(upd, vmin, best_v),
                jnp.where(upd, imin, best_i))

    v0 = jnp.full((BM, 1), jnp.inf, jnp.float32)
    i0 = jnp.zeros((BM, 1), jnp.float32)
    _, best_i = lax.fori_loop(0, N_CODES // BN, chunk, (v0, i0),
                              unroll=2)
    idx_ref[...] = best_i.astype(jnp.int32)


_argmin_call = pl.pallas_call(
    _argmin_body,
    grid=(N_TOK // BM,),
    in_specs=[
        pl.BlockSpec((BM, D), lambda i: (i, 0)),
        pl.BlockSpec((D, N_CODES), lambda i: (0, 0)),
        pl.BlockSpec((BM, 1), lambda i: (i, 0)),
        pl.BlockSpec((1, N_CODES), lambda i: (0, 0)),
    ],
    out_specs=pl.BlockSpec((BM, 1), lambda i: (i, 0)),
    out_shape=jax.ShapeDtypeStruct((N_TOK, 1), jnp.int32),
)


@functools.cache
def _make_sc_gather():
    info = plsc.get_sparse_core_info()
    nw = info.num_cores * info.num_subcores
    bpw = N_TOK // nw
    mesh = plsc.VectorSubcoreMesh(core_axis_name="c", subcore_axis_name="s")

    @functools.partial(
        pl.kernel, mesh=mesh,
        out_type=jax.ShapeDtypeStruct((N_TOK, D), jnp.float32),
        scratch_types=[
            pltpu.VMEM((bpw,), jnp.int32),
            pltpu.VMEM((bpw, D), jnp.float32),
            pltpu.SemaphoreType.DMA,
        ],
    )
    def gather(table_hbm, idx_hbm, out_hbm, idx_v, rows_v, sem):
        wid = lax.axis_index("s") * info.num_cores + lax.axis_index("c")
        base = wid * bpw
        pltpu.sync_copy(idx_hbm.at[pl.ds(base, bpw)], idx_v)
        pltpu.async_copy(table_hbm.at[idx_v], rows_v, sem).wait()
        pltpu.sync_copy(rows_v, out_hbm.at[pl.ds(base, bpw)])

    return gather


def _st_body(flat_ref, q_ref, st_ref, loss_ref, acc_ref):
    i = pl.program_id(0)
    x = flat_ref[...]
    q = q_ref[...]
    dif = q - x
    st_ref[...] = x + dif
    part = jnp.sum(dif * dif)

    @pl.when(i == 0)
    def _():
        acc_ref[0] = part

    @pl.when(i > 0)
    def _():
        acc_ref[0] = acc_ref[0] + part

    @pl.when(i == pl.num_programs(0) - 1)
    def _():
        loss_ref[0, 0] = acc_ref[0] * ((1.0 + C_COST) / (N_TOK * D))


_st_call = pl.pallas_call(
    _st_body,
    grid=(N_TOK // BM3,),
    in_specs=[
        pl.BlockSpec((BM3, D), lambda i: (i, 0)),
        pl.BlockSpec((BM3, D), lambda i: (i, 0)),
    ],
    out_specs=[
        pl.BlockSpec((BM3, D), lambda i: (i, 0)),
        pl.BlockSpec(memory_space=pltpu.SMEM),
    ],
    out_shape=[
        jax.ShapeDtypeStruct((N_TOK, D), jnp.float32),
        jax.ShapeDtypeStruct((1, 1), jnp.float32),
    ],
    scratch_shapes=[pltpu.SMEM((1,), jnp.float32)],
)


def kernel(inputs, W):
    x = jnp.transpose(inputs, (0, 2, 3, 4, 1))
    flat = x.reshape(-1, D)
    sf = jnp.sum(flat ** 2, axis=1, keepdims=True)
    sw = jnp.sum(W ** 2, axis=1).reshape(1, N_CODES)
    idx = _argmin_call(flat * 2.0, W.T, sf, sw)
    q = _make_sc_gather()(W, idx.reshape(-1))
    st, loss = _st_call(flat, q)
    out = jnp.transpose(st.reshape(2, 16, 16, 16, D), (0, 4, 1, 2, 3))
    return (loss[0, 0], out)


# speedup vs baseline: 2.0088x; 1.0606x over previous
import functools

import jax
import jax.numpy as jnp
from jax import lax
from jax.experimental import pallas as pl
from jax.experimental.pallas import tpu as pltpu
from jax.experimental.pallas import tpu_sc as plsc

N_CODES = 8192
N_TOK = 8192
D = 256
C_COST = 0.25

BM = 512
BN = 1024
BM3 = 512


def _argmin_body(f_ref, wt_ref, sf_ref, sw_ref, idx_ref):
    f2 = f_ref[...] * 2.0
    sf = sf_ref[...]
    cols = lax.broadcasted_iota(jnp.int32, (1, BN), 1).astype(jnp.float32)

    def chunk(t, carry):
        best_v, best_i = carry
        w = wt_ref[:, pl.ds(t * BN, BN)]
        mm2 = lax.dot_general(f2, w, (((1,), (0,)), ((), ())),
                              preferred_element_type=jnp.float32)
        sw = sw_ref[:, pl.ds(t * BN, BN)]
        d = (sf + sw) - mm2
        vmin = jnp.min(d, axis=1, keepdims=True)
        imin = jnp.min(jnp.where(d == vmin, cols, jnp.float32(jnp.inf)),
                       axis=1, keepdims=True)
        imin = imin + jnp.float32(BN) * t.astype(jnp.float32)
        upd = vmin < best_v
        return (jnp.where(upd, vmin, best_v),
                jnp.where(upd, imin, best_i))

    v0 = jnp.full((BM, 1), jnp.inf, jnp.float32)
    i0 = jnp.zeros((BM, 1), jnp.float32)
    _, best_i = lax.fori_loop(0, N_CODES // BN, chunk, (v0, i0),
                              unroll=2)
    idx_ref[...] = best_i.astype(jnp.int32)


_argmin_call = pl.pallas_call(
    _argmin_body,
    grid=(N_TOK // BM,),
    in_specs=[
        pl.BlockSpec((BM, D), lambda i: (i, 0)),
        pl.BlockSpec((D, N_CODES), lambda i: (0, 0)),
        pl.BlockSpec((BM, 1), lambda i: (i, 0)),
        pl.BlockSpec((1, N_CODES), lambda i: (0, 0)),
    ],
    out_specs=pl.BlockSpec((BM, 1), lambda i: (i, 0)),
    out_shape=jax.ShapeDtypeStruct((N_TOK, 1), jnp.int32),
)


@functools.cache
def _make_sc_gather():
    info = plsc.get_sparse_core_info()
    nw = info.num_cores * info.num_subcores
    bpw = N_TOK // nw
    mesh = plsc.VectorSubcoreMesh(core_axis_name="c", subcore_axis_name="s")

    @functools.partial(
        pl.kernel, mesh=mesh,
        out_type=jax.ShapeDtypeStruct((N_TOK, D), jnp.float32),
        scratch_types=[
            pltpu.VMEM((bpw,), jnp.int32),
            pltpu.VMEM((bpw, D), jnp.float32),
            pltpu.SemaphoreType.DMA,
        ],
    )
    def gather(table_hbm, idx_hbm, out_hbm, idx_v, rows_v, sem):
        wid = lax.axis_index("s") * info.num_cores + lax.axis_index("c")
        base = wid * bpw
        pltpu.sync_copy(idx_hbm.at[pl.ds(base, bpw)], idx_v)
        pltpu.async_copy(table_hbm.at[idx_v], rows_v, sem).wait()
        pltpu.sync_copy(rows_v, out_hbm.at[pl.ds(base, bpw)])

    return gather


def _st_body(flat_ref, q_ref, st_ref, loss_ref, acc_ref):
    i = pl.program_id(0)
    x = flat_ref[...]
    q = q_ref[...]
    dif = q - x
    st = x + dif
    st_ref[...] = jnp.transpose(st, (1, 0)).reshape(1, D, BM3)
    part = jnp.sum(dif * dif)

    @pl.when(i == 0)
    def _():
        acc_ref[0] = part

    @pl.when(i > 0)
    def _():
        acc_ref[0] = acc_ref[0] + part

    @pl.when(i == pl.num_programs(0) - 1)
    def _():
        loss_ref[0, 0] = acc_ref[0] * ((1.0 + C_COST) / (N_TOK * D))


_st_call = pl.pallas_call(
    _st_body,
    grid=(N_TOK // BM3,),
    in_specs=[
        pl.BlockSpec((BM3, D), lambda i: (i, 0)),
        pl.BlockSpec((BM3, D), lambda i: (i, 0)),
    ],
    out_specs=[
        pl.BlockSpec((1, D, BM3), lambda i: (i // (4096 // BM3), 0,
                                             i % (4096 // BM3))),
        pl.BlockSpec(memory_space=pltpu.SMEM),
    ],
    out_shape=[
        jax.ShapeDtypeStruct((2, D, 4096), jnp.float32),
        jax.ShapeDtypeStruct((1, 1), jnp.float32),
    ],
    scratch_shapes=[pltpu.SMEM((1,), jnp.float32)],
)


def kernel(inputs, W):
    x = jnp.transpose(inputs, (0, 2, 3, 4, 1))
    flat = x.reshape(-1, D)
    sf = jnp.sum(flat ** 2, axis=1, keepdims=True)
    sw = jnp.sum(W ** 2, axis=1).reshape(1, N_CODES)
    idx = _argmin_call(flat, W.T, sf, sw)
    q = _make_sc_gather()(W, idx.reshape(-1))
    st, loss = _st_call(flat, q)
    out = st.reshape(2, D, 16, 16, 16)
    return (loss[0, 0], out)


# speedup vs baseline: 2.0472x; 1.0191x over previous
import functools

import jax
import jax.numpy as jnp
from jax import lax
from jax.experimental import pallas as pl
from jax.experimental.pallas import tpu as pltpu
from jax.experimental.pallas import tpu_sc as plsc

N_CODES = 8192
N_TOK = 8192
D = 256
C_COST = 0.25

BM = 512
BN = 1024
BM3 = 512


def _argmin_body(x_ref, w_ref, sf_ref, sw_ref, idx_ref):
    f2t = x_ref[0] * 2.0
    sf = sf_ref[0]
    rows = lax.broadcasted_iota(jnp.int32, (BN, 1), 0).astype(jnp.float32)

    def chunk(t, carry):
        best_v, best_i = carry
        w = w_ref[pl.ds(t * BN, BN), :]
        mm2 = lax.dot_general(w, f2t, (((1,), (0,)), ((), ())),
                              preferred_element_type=jnp.float32)
        sw = sw_ref[pl.ds(t * BN, BN), :]
        d = (sf + sw) - mm2
        vmin = jnp.min(d, axis=0, keepdims=True)
        imin = jnp.min(jnp.where(d == vmin, rows, jnp.float32(jnp.inf)),
                       axis=0, keepdims=True)
        imin = imin + jnp.float32(BN) * t.astype(jnp.float32)
        upd = vmin < best_v
        return (jnp.where(upd, vmin, best_v),
                jnp.where(upd, imin, best_i))

    v0 = jnp.full((1, BM), jnp.inf, jnp.float32)
    i0 = jnp.zeros((1, BM), jnp.float32)
    _, best_i = lax.fori_loop(0, N_CODES // BN, chunk, (v0, i0),
                              unroll=2)
    idx_ref[...] = best_i.astype(jnp.int32).reshape(1, 1, BM)


_argmin_call = pl.pallas_call(
    _argmin_body,
    grid=(N_TOK // BM,),
    in_specs=[
        pl.BlockSpec((1, D, BM), lambda i: (i // (4096 // BM), 0,
                                            i % (4096 // BM))),
        pl.BlockSpec((N_CODES, D), lambda i: (0, 0)),
        pl.BlockSpec((1, 1, BM), lambda i: (i, 0, 0)),
        pl.BlockSpec((N_CODES, 1), lambda i: (0, 0)),
    ],
    out_specs=pl.BlockSpec((1, 1, BM), lambda i: (i, 0, 0)),
    out_shape=jax.ShapeDtypeStruct((N_TOK // BM, 1, BM), jnp.int32),
)


@functools.cache
def _make_sc_gather():
    info = plsc.get_sparse_core_info()
    nw = info.num_cores * info.num_subcores
    bpw = N_TOK // nw
    mesh = plsc.VectorSubcoreMesh(core_axis_name="c", subcore_axis_name="s")

    @functools.partial(
        pl.kernel, mesh=mesh,
        out_type=jax.ShapeDtypeStruct((N_TOK, D), jnp.float32),
        scratch_types=[
            pltpu.VMEM((bpw,), jnp.int32),
            pltpu.VMEM((bpw, D), jnp.float32),
            pltpu.SemaphoreType.DMA,
        ],
    )
    def gather(table_hbm, idx_hbm, out_hbm, idx_v, rows_v, sem):
        wid = lax.axis_index("s") * info.num_cores + lax.axis_index("c")
        base = wid * bpw
        pltpu.sync_copy(idx_hbm.at[pl.ds(base, bpw)], idx_v)
        pltpu.async_copy(table_hbm.at[idx_v], rows_v, sem).wait()
        pltpu.sync_copy(rows_v, out_hbm.at[pl.ds(base, bpw)])

    return gather


def _st_body(x_ref, q_ref, st_ref, loss_ref, acc_ref):
    i = pl.program_id(0)
    xt = x_ref[0]
    qt = jnp.transpose(q_ref[...], (1, 0))
    dif = qt - xt
    st_ref[...] = (xt + dif).reshape(1, D, BM3)
    part = jnp.sum(dif * dif)

    @pl.when(i == 0)
    def _():
        acc_ref[0] = part

    @pl.when(i > 0)
    def _():
        acc_ref[0] = acc_ref[0] + part

    @pl.when(i == pl.num_programs(0) - 1)
    def _():
        loss_ref[0, 0] = acc_ref[0] * ((1.0 + C_COST) / (N_TOK * D))


_st_call = pl.pallas_call(
    _st_body,
    grid=(N_TOK // BM3,),
    in_specs=[
        pl.BlockSpec((1, D, BM3), lambda i: (i // (4096 // BM3), 0,
                                             i % (4096 // BM3))),
        pl.BlockSpec((BM3, D), lambda i: (i, 0)),
    ],
    out_specs=[
        pl.BlockSpec((1, D, BM3), lambda i: (i // (4096 // BM3), 0,
                                             i % (4096 // BM3))),
        pl.BlockSpec(memory_space=pltpu.SMEM),
    ],
    out_shape=[
        jax.ShapeDtypeStruct((2, D, 4096), jnp.float32),
        jax.ShapeDtypeStruct((1, 1), jnp.float32),
    ],
    scratch_shapes=[pltpu.SMEM((1,), jnp.float32)],
)


def kernel(inputs, W):
    xr = inputs.reshape(2, D, 4096)
    x = jnp.transpose(inputs, (0, 2, 3, 4, 1))
    flat = x.reshape(-1, D)
    sf = jnp.sum(flat ** 2, axis=1, keepdims=True).reshape(N_TOK // BM, 1, BM)
    sw = jnp.sum(W ** 2, axis=1, keepdims=True)
    idx3 = _argmin_call(xr, W, sf, sw)
    q = _make_sc_gather()(W, idx3.reshape(-1))
    st, loss = _st_call(xr, q)
    out = st.reshape(2, D, 16, 16, 16)
    return (loss[0, 0], out)


# speedup vs baseline: 2.4831x; 1.2129x over previous
import functools

import jax
import jax.numpy as jnp
from jax import lax
from jax.experimental import pallas as pl
from jax.experimental.pallas import tpu as pltpu
from jax.experimental.pallas import tpu_sc as plsc

N_CODES = 8192
N_TOK = 8192
D = 256
C_COST = 0.25

BM = 512
BN = 1024
BM3 = 512


def _argmin_body(x_ref, w_ref, sf_ref, sw_ref, idx_ref):
    f2t = x_ref[0] * 2.0
    sf = sf_ref[0]
    s_iota = lax.broadcasted_iota(jnp.int32, (8, 1), 0).astype(jnp.float32)

    def chunk(t, carry):
        best_v, best_i = carry
        w = w_ref[pl.ds(t * BN, BN), :]
        mm2 = lax.dot_general(w, f2t, (((1,), (0,)), ((), ())),
                              preferred_element_type=jnp.float32)
        sw = sw_ref[pl.ds(t * BN, BN), :]
        run_v = jnp.full((8, BM), jnp.inf, jnp.float32)
        run_g = jnp.zeros((8, BM), jnp.float32)
        for g in range(BN // 8):
            d_g = (sf + sw[g * 8:(g + 1) * 8, :]) - mm2[g * 8:(g + 1) * 8, :]
            upd_g = d_g < run_v
            run_g = jnp.where(upd_g, jnp.float32(g), run_g)
            run_v = jnp.where(upd_g, d_g, run_v)
        vmin = jnp.min(run_v, axis=0, keepdims=True)
        glob = run_g * 8.0 + s_iota
        imin = jnp.min(jnp.where(run_v == vmin, glob, jnp.float32(jnp.inf)),
                       axis=0, keepdims=True)
        imin = imin + jnp.float32(BN) * t.astype(jnp.float32)
        upd = vmin < best_v
        return (jnp.where(upd, vmin, best_v),
                jnp.where(upd, imin, best_i))

    v0 = jnp.full((1, BM), jnp.inf, jnp.float32)
    i0 = jnp.zeros((1, BM), jnp.float32)
    _, best_i = lax.fori_loop(0, N_CODES // BN, chunk, (v0, i0),
                              unroll=2)
    idx_ref[...] = best_i.astype(jnp.int32).reshape(1, 1, BM)


_argmin_call = pl.pallas_call(
    _argmin_body,
    grid=(N_TOK // BM,),
    in_specs=[
        pl.BlockSpec((1, D, BM), lambda i: (i // (4096 // BM), 0,
                                            i % (4096 // BM))),
        pl.BlockSpec((N_CODES, D), lambda i: (0, 0)),
        pl.BlockSpec((1, 1, BM), lambda i: (i, 0, 0)),
        pl.BlockSpec((N_CODES, 1), lambda i: (0, 0)),
    ],
    out_specs=pl.BlockSpec((1, 1, BM), lambda i: (i, 0, 0)),
    out_shape=jax.ShapeDtypeStruct((N_TOK // BM, 1, BM), jnp.int32),
)


@functools.cache
def _make_sc_gather():
    info = plsc.get_sparse_core_info()
    nw = info.num_cores * info.num_subcores
    bpw = N_TOK // nw
    mesh = plsc.VectorSubcoreMesh(core_axis_name="c", subcore_axis_name="s")

    @functools.partial(
        pl.kernel, mesh=mesh,
        out_type=jax.ShapeDtypeStruct((N_TOK, D), jnp.float32),
        scratch_types=[
            pltpu.VMEM((bpw,), jnp.int32),
            pltpu.VMEM((bpw, D), jnp.float32),
            pltpu.SemaphoreType.DMA,
        ],
    )
    def gather(table_hbm, idx_hbm, out_hbm, idx_v, rows_v, sem):
        wid = lax.axis_index("s") * info.num_cores + lax.axis_index("c")
        base = wid * bpw
        pltpu.sync_copy(idx_hbm.at[pl.ds(base, bpw)], idx_v)
        pltpu.async_copy(table_hbm.at[idx_v], rows_v, sem).wait()
        pltpu.sync_copy(rows_v, out_hbm.at[pl.ds(base, bpw)])

    return gather


def _st_body(x_ref, q_ref, st_ref, loss_ref, acc_ref):
    i = pl.program_id(0)
    xt = x_ref[0]
    qt = jnp.transpose(q_ref[...], (1, 0))
    dif = qt - xt
    st_ref[...] = (xt + dif).reshape(1, D, BM3)
    part = jnp.sum(dif * dif)

    @pl.when(i == 0)
    def _():
        acc_ref[0] = part

    @pl.when(i > 0)
    def _():
        acc_ref[0] = acc_ref[0] + part

    @pl.when(i == pl.num_programs(0) - 1)
    def _():
        loss_ref[0, 0] = acc_ref[0] * ((1.0 + C_COST) / (N_TOK * D))


_st_call = pl.pallas_call(
    _st_body,
    grid=(N_TOK // BM3,),
    in_specs=[
        pl.BlockSpec((1, D, BM3), lambda i: (i // (4096 // BM3), 0,
                                             i % (4096 // BM3))),
        pl.BlockSpec((BM3, D), lambda i: (i, 0)),
    ],
    out_specs=[
        pl.BlockSpec((1, D, BM3), lambda i: (i // (4096 // BM3), 0,
                                             i % (4096 // BM3))),
        pl.BlockSpec(memory_space=pltpu.SMEM),
    ],
    out_shape=[
        jax.ShapeDtypeStruct((2, D, 4096), jnp.float32),
        jax.ShapeDtypeStruct((1, 1), jnp.float32),
    ],
    scratch_shapes=[pltpu.SMEM((1,), jnp.float32)],
)


def kernel(inputs, W):
    xr = inputs.reshape(2, D, 4096)
    x = jnp.transpose(inputs, (0, 2, 3, 4, 1))
    flat = x.reshape(-1, D)
    sf = jnp.sum(flat ** 2, axis=1, keepdims=True).reshape(N_TOK // BM, 1, BM)
    sw = jnp.sum(W ** 2, axis=1, keepdims=True)
    idx3 = _argmin_call(xr, W, sf, sw)
    q = _make_sc_gather()(W, idx3.reshape(-1))
    st, loss = _st_call(xr, q)
    out = st.reshape(2, D, 16, 16, 16)
    return (loss[0, 0], out)


# speedup vs baseline: 2.5498x; 1.0269x over previous
import functools

import jax
import jax.numpy as jnp
from jax import lax
from jax.experimental import pallas as pl
from jax.experimental.pallas import tpu as pltpu
from jax.experimental.pallas import tpu_sc as plsc

N_CODES = 8192
N_TOK = 8192
D = 256
C_COST = 0.25

BM = 512
BN = 1024
BM3 = 512


def _argmin_body(x_ref, w_ref, sw_ref, idx_ref):
    xt = x_ref[0]
    f2t = xt * 2.0
    sf = jnp.sum(xt * xt, axis=0, keepdims=True)
    s_iota = lax.broadcasted_iota(jnp.int32, (8, 1), 0).astype(jnp.float32)

    def chunk(t, carry):
        best_v, best_i = carry
        w = w_ref[pl.ds(t * BN, BN), :]
        mm2 = lax.dot_general(w, f2t, (((1,), (0,)), ((), ())),
                              preferred_element_type=jnp.float32)
        sw = sw_ref[pl.ds(t * BN, BN), :]
        run_v = jnp.full((8, BM), jnp.inf, jnp.float32)
        run_g = jnp.zeros((8, BM), jnp.float32)
        for g in range(BN // 8):
            d_g = (sf + sw[g * 8:(g + 1) * 8, :]) - mm2[g * 8:(g + 1) * 8, :]
            upd_g = d_g < run_v
            run_g = jnp.where(upd_g, jnp.float32(g), run_g)
            run_v = jnp.where(upd_g, d_g, run_v)
        vmin = jnp.min(run_v, axis=0, keepdims=True)
        glob = run_g * 8.0 + s_iota
        imin = jnp.min(jnp.where(run_v == vmin, glob, jnp.float32(jnp.inf)),
                       axis=0, keepdims=True)
        imin = imin + jnp.float32(BN) * t.astype(jnp.float32)
        upd = vmin < best_v
        return (jnp.where(upd, vmin, best_v),
                jnp.where(upd, imin, best_i))

    v0 = jnp.full((1, BM), jnp.inf, jnp.float32)
    i0 = jnp.zeros((1, BM), jnp.float32)
    _, best_i = lax.fori_loop(0, N_CODES // BN, chunk, (v0, i0),
                              unroll=2)
    idx_ref[...] = best_i.astype(jnp.int32).reshape(1, 1, BM)


_argmin_call = pl.pallas_call(
    _argmin_body,
    grid=(N_TOK // BM,),
    in_specs=[
        pl.BlockSpec((1, D, BM), lambda i: (i // (4096 // BM), 0,
                                            i % (4096 // BM))),
        pl.BlockSpec((N_CODES, D), lambda i: (0, 0)),
        pl.BlockSpec((N_CODES, 1), lambda i: (0, 0)),
    ],
    out_specs=pl.BlockSpec((1, 1, BM), lambda i: (i, 0, 0)),
    out_shape=jax.ShapeDtypeStruct((N_TOK // BM, 1, BM), jnp.int32),
)


@functools.cache
def _make_sc_gather():
    info = plsc.get_sparse_core_info()
    nw = info.num_cores * info.num_subcores
    bpw = N_TOK // nw
    mesh = plsc.VectorSubcoreMesh(core_axis_name="c", subcore_axis_name="s")

    @functools.partial(
        pl.kernel, mesh=mesh,
        out_type=jax.ShapeDtypeStruct((N_TOK, D), jnp.float32),
        scratch_types=[
            pltpu.VMEM((bpw,), jnp.int32),
            pltpu.VMEM((bpw, D), jnp.float32),
            pltpu.SemaphoreType.DMA,
        ],
    )
    def gather(table_hbm, idx_hbm, out_hbm, idx_v, rows_v, sem):
        wid = lax.axis_index("s") * info.num_cores + lax.axis_index("c")
        base = wid * bpw
        pltpu.sync_copy(idx_hbm.at[pl.ds(base, bpw)], idx_v)
        pltpu.async_copy(table_hbm.at[idx_v], rows_v, sem).wait()
        pltpu.sync_copy(rows_v, out_hbm.at[pl.ds(base, bpw)])

    return gather


def _st_body(x_ref, q_ref, st_ref, loss_ref, acc_ref):
    i = pl.program_id(0)
    xt = x_ref[0]
    qt = jnp.transpose(q_ref[...], (1, 0))
    dif = qt - xt
    st_ref[...] = (xt + dif).reshape(1, D, BM3)
    part = jnp.sum(dif * dif)

    @pl.when(i == 0)
    def _():
        acc_ref[0] = part

    @pl.when(i > 0)
    def _():
        acc_ref[0] = acc_ref[0] + part

    @pl.when(i == pl.num_programs(0) - 1)
    def _():
        loss_ref[0, 0] = acc_ref[0] * ((1.0 + C_COST) / (N_TOK * D))


_st_call = pl.pallas_call(
    _st_body,
    grid=(N_TOK // BM3,),
    in_specs=[
        pl.BlockSpec((1, D, BM3), lambda i: (i // (4096 // BM3), 0,
                                             i % (4096 // BM3))),
        pl.BlockSpec((BM3, D), lambda i: (i, 0)),
    ],
    out_specs=[
        pl.BlockSpec((1, D, BM3), lambda i: (i // (4096 // BM3), 0,
                                             i % (4096 // BM3))),
        pl.BlockSpec(memory_space=pltpu.SMEM),
    ],
    out_shape=[
        jax.ShapeDtypeStruct((2, D, 4096), jnp.float32),
        jax.ShapeDtypeStruct((1, 1), jnp.float32),
    ],
    scratch_shapes=[pltpu.SMEM((1,), jnp.float32)],
)


def kernel(inputs, W):
    xr = inputs.reshape(2, D, 4096)
    sw = jnp.sum(W ** 2, axis=1, keepdims=True)
    idx3 = _argmin_call(xr, W, sw)
    q = _make_sc_gather()(W, idx3.reshape(-1))
    st, loss = _st_call(xr, q)
    out = st.reshape(2, D, 16, 16, 16)
    return (loss[0, 0], out)


# speedup vs baseline: 2.6905x; 1.0552x over previous
import functools

import jax
import jax.numpy as jnp
from jax import lax
from jax.experimental import pallas as pl
from jax.experimental.pallas import tpu as pltpu
from jax.experimental.pallas import tpu_sc as plsc

N_CODES = 8192
N_TOK = 8192
D = 256
C_COST = 0.25

BM = 512
BN = 1024
BM3 = 512


def _argmin_body(x_ref, w_ref, idx_ref, sw_ref):
    @pl.when(pl.program_id(0) == 0)
    def _():
        wv = w_ref[...]
        sw_ref[...] = jnp.sum(wv * wv, axis=1, keepdims=True)

    xt = x_ref[0]
    f2t = xt * 2.0
    sf = jnp.sum(xt * xt, axis=0, keepdims=True)
    s_iota = lax.broadcasted_iota(jnp.int32, (8, 1), 0).astype(jnp.float32)

    def chunk(t, carry):
        best_v, best_i = carry
        w = w_ref[pl.ds(t * BN, BN), :]
        mm2 = lax.dot_general(w, f2t, (((1,), (0,)), ((), ())),
                              preferred_element_type=jnp.float32)
        sw = sw_ref[pl.ds(t * BN, BN), :]
        run_v = jnp.full((8, BM), jnp.inf, jnp.float32)
        run_g = jnp.zeros((8, BM), jnp.float32)
        for g in range(BN // 8):
            d_g = (sf + sw[g * 8:(g + 1) * 8, :]) - mm2[g * 8:(g + 1) * 8, :]
            upd_g = d_g < run_v
            run_g = jnp.where(upd_g, jnp.float32(g), run_g)
            run_v = jnp.where(upd_g, d_g, run_v)
        vmin = jnp.min(run_v, axis=0, keepdims=True)
        glob = run_g * 8.0 + s_iota
        imin = jnp.min(jnp.where(run_v == vmin, glob, jnp.float32(jnp.inf)),
                       axis=0, keepdims=True)
        imin = imin + jnp.float32(BN) * t.astype(jnp.float32)
        upd = vmin < best_v
        return (jnp.where(upd, vmin, best_v),
                jnp.where(upd, imin, best_i))

    v0 = jnp.full((1, BM), jnp.inf, jnp.float32)
    i0 = jnp.zeros((1, BM), jnp.float32)
    _, best_i = lax.fori_loop(0, N_CODES // BN, chunk, (v0, i0),
                              unroll=2)
    idx_ref[...] = best_i.astype(jnp.int32).reshape(1, 1, BM)


_argmin_call = pl.pallas_call(
    _argmin_body,
    grid=(N_TOK // BM,),
    in_specs=[
        pl.BlockSpec((1, D, BM), lambda i: (i // (4096 // BM), 0,
                                            i % (4096 // BM))),
        pl.BlockSpec((N_CODES, D), lambda i: (0, 0)),
    ],
    out_specs=pl.BlockSpec((1, 1, BM), lambda i: (i, 0, 0)),
    out_shape=jax.ShapeDtypeStruct((N_TOK // BM, 1, BM), jnp.int32),
    scratch_shapes=[pltpu.VMEM((N_CODES, 1), jnp.float32)],
)


@functools.cache
def _make_sc_gather():
    info = plsc.get_sparse_core_info()
    nw = info.num_cores * info.num_subcores
    bpw = N_TOK // nw
    mesh = plsc.VectorSubcoreMesh(core_axis_name="c", subcore_axis_name="s")

    @functools.partial(
        pl.kernel, mesh=mesh,
        out_type=jax.ShapeDtypeStruct((N_TOK, D), jnp.float32),
        scratch_types=[
            pltpu.VMEM((bpw,), jnp.int32),
            pltpu.VMEM((bpw, D), jnp.float32),
            pltpu.SemaphoreType.DMA,
        ],
    )
    def gather(table_hbm, idx_hbm, out_hbm, idx_v, rows_v, sem):
        wid = lax.axis_index("s") * info.num_cores + lax.axis_index("c")
        base = wid * bpw
        pltpu.sync_copy(idx_hbm.at[pl.ds(base, bpw)], idx_v)
        pltpu.async_copy(table_hbm.at[idx_v], rows_v, sem).wait()
        pltpu.sync_copy(rows_v, out_hbm.at[pl.ds(base, bpw)])

    return gather


def _st_body(x_ref, q_ref, st_ref, loss_ref, acc_ref):
    i = pl.program_id(0)
    xt = x_ref[0]
    qt = jnp.transpose(q_ref[...], (1, 0))
    dif = qt - xt
    st_ref[...] = (xt + dif).reshape(1, D, BM3)
    part = jnp.sum(dif * dif)

    @pl.when(i == 0)
    def _():
        acc_ref[0] = part

    @pl.when(i > 0)
    def _():
        acc_ref[0] = acc_ref[0] + part

    @pl.when(i == pl.num_programs(0) - 1)
    def _():
        loss_ref[0, 0] = acc_ref[0] * ((1.0 + C_COST) / (N_TOK * D))


_st_call = pl.pallas_call(
    _st_body,
    grid=(N_TOK // BM3,),
    in_specs=[
        pl.BlockSpec((1, D, BM3), lambda i: (i // (4096 // BM3), 0,
                                             i % (4096 // BM3))),
        pl.BlockSpec((BM3, D), lambda i: (i, 0)),
    ],
    out_specs=[
        pl.BlockSpec((1, D, BM3), lambda i: (i // (4096 // BM3), 0,
                                             i % (4096 // BM3))),
        pl.BlockSpec(memory_space=pltpu.SMEM),
    ],
    out_shape=[
        jax.ShapeDtypeStruct((2, D, 4096), jnp.float32),
        jax.ShapeDtypeStruct((1, 1), jnp.float32),
    ],
    scratch_shapes=[pltpu.SMEM((1,), jnp.float32)],
)


def kernel(inputs, W):
    xr = inputs.reshape(2, D, 4096)
    idx3 = _argmin_call(xr, W)
    q = _make_sc_gather()(W, idx3.reshape(-1))
    st, loss = _st_call(xr, q)
    out = st.reshape(2, D, 16, 16, 16)
    return (loss[0, 0], out)


# speedup vs baseline: 3.0603x; 1.1374x over previous
import functools

import jax
import jax.numpy as jnp
from jax import lax
from jax.experimental import pallas as pl
from jax.experimental.pallas import tpu as pltpu
from jax.experimental.pallas import tpu_sc as plsc

N_CODES = 8192
N_TOK = 8192
D = 256
C_COST = 0.25

BM = 1024
BN = 4096
BM3 = 512


def _argmin_body(x_ref, w_ref, idx_ref, sw_ref):
    @pl.when(pl.program_id(0) == 0)
    def _():
        wv = w_ref[...]
        sw_ref[...] = jnp.sum(wv * wv, axis=1, keepdims=True)

    xt = x_ref[0]
    f2t = xt * 2.0
    sf = jnp.sum(xt * xt, axis=0, keepdims=True)
    s_iota = lax.broadcasted_iota(jnp.int32, (8, 1), 0).astype(jnp.float32)

    def chunk(t, carry):
        best_v, best_i = carry
        w = w_ref[pl.ds(t * BN, BN), :]
        mm2 = lax.dot_general(w, f2t, (((1,), (0,)), ((), ())),
                              preferred_element_type=jnp.float32)
        sw = sw_ref[pl.ds(t * BN, BN), :]
        n_g = BN // 8
        accs = []
        for k in range(4):
            acc_v = jnp.full((8, BM), jnp.inf, jnp.float32)
            acc_g = jnp.zeros((8, BM), jnp.float32)
            for g in range(k * n_g // 4, (k + 1) * n_g // 4):
                d_g = ((sf + sw[g * 8:(g + 1) * 8, :])
                       - mm2[g * 8:(g + 1) * 8, :])
                upd_g = d_g < acc_v
                acc_g = jnp.where(upd_g, jnp.float32(g), acc_g)
                acc_v = jnp.where(upd_g, d_g, acc_v)
            accs.append((acc_v, acc_g))
        run_v, run_g = accs[0]
        for acc_v, acc_g in accs[1:]:
            upd_k = acc_v < run_v
            run_g = jnp.where(upd_k, acc_g, run_g)
            run_v = jnp.where(upd_k, acc_v, run_v)
        vmin = jnp.min(run_v, axis=0, keepdims=True)
        glob = run_g * 8.0 + s_iota
        imin = jnp.min(jnp.where(run_v == vmin, glob, jnp.float32(jnp.inf)),
                       axis=0, keepdims=True)
        imin = imin + jnp.float32(BN) * t.astype(jnp.float32)
        upd = vmin < best_v
        return (jnp.where(upd, vmin, best_v),
                jnp.where(upd, imin, best_i))

    v0 = jnp.full((1, BM), jnp.inf, jnp.float32)
    i0 = jnp.zeros((1, BM), jnp.float32)
    _, best_i = lax.fori_loop(0, N_CODES // BN, chunk, (v0, i0),
                              unroll=2)
    idx_ref[...] = best_i.astype(jnp.int32).reshape(1, 1, BM)


_argmin_call = pl.pallas_call(
    _argmin_body,
    grid=(N_TOK // BM,),
    in_specs=[
        pl.BlockSpec((1, D, BM), lambda i: (i // (4096 // BM), 0,
                                            i % (4096 // BM))),
        pl.BlockSpec((N_CODES, D), lambda i: (0, 0)),
    ],
    out_specs=pl.BlockSpec((1, 1, BM), lambda i: (i, 0, 0)),
    out_shape=jax.ShapeDtypeStruct((N_TOK // BM, 1, BM), jnp.int32),
    scratch_shapes=[pltpu.VMEM((N_CODES, 1), jnp.float32)],
)


@functools.cache
def _make_sc_gather():
    info = plsc.get_sparse_core_info()
    nw = info.num_cores * info.num_subcores
    bpw = N_TOK // nw
    mesh = plsc.VectorSubcoreMesh(core_axis_name="c", subcore_axis_name="s")

    @functools.partial(
        pl.kernel, mesh=mesh,
        out_type=jax.ShapeDtypeStruct((N_TOK, D), jnp.float32),
        scratch_types=[
            pltpu.VMEM((bpw,), jnp.int32),
            pltpu.VMEM((bpw, D), jnp.float32),
            pltpu.SemaphoreType.DMA,
        ],
    )
    def gather(table_hbm, idx_hbm, out_hbm, idx_v, rows_v, sem):
        wid = lax.axis_index("s") * info.num_cores + lax.axis_index("c")
        base = wid * bpw
        pltpu.sync_copy(idx_hbm.at[pl.ds(base, bpw)], idx_v)
        pltpu.async_copy(table_hbm.at[idx_v], rows_v, sem).wait()
        pltpu.sync_copy(rows_v, out_hbm.at[pl.ds(base, bpw)])

    return gather


def _st_body(x_ref, q_ref, st_ref, loss_ref, acc_ref):
    i = pl.program_id(0)
    xt = x_ref[0]
    qt = jnp.transpose(q_ref[...], (1, 0))
    dif = qt - xt
    st_ref[...] = (xt + dif).reshape(1, D, BM3)
    part = jnp.sum(dif * dif)

    @pl.when(i == 0)
    def _():
        acc_ref[0] = part

    @pl.when(i > 0)
    def _():
        acc_ref[0] = acc_ref[0] + part

    @pl.when(i == pl.num_programs(0) - 1)
    def _():
        loss_ref[0, 0] = acc_ref[0] * ((1.0 + C_COST) / (N_TOK * D))


_st_call = pl.pallas_call(
    _st_body,
    grid=(N_TOK // BM3,),
    in_specs=[
        pl.BlockSpec((1, D, BM3), lambda i: (i // (4096 // BM3), 0,
                                             i % (4096 // BM3))),
        pl.BlockSpec((BM3, D), lambda i: (i, 0)),
    ],
    out_specs=[
        pl.BlockSpec((1, D, BM3), lambda i: (i // (4096 // BM3), 0,
                                             i % (4096 // BM3))),
        pl.BlockSpec(memory_space=pltpu.SMEM),
    ],
    out_shape=[
        jax.ShapeDtypeStruct((2, D, 4096), jnp.float32),
        jax.ShapeDtypeStruct((1, 1), jnp.float32),
    ],
    scratch_shapes=[pltpu.SMEM((1,), jnp.float32)],
)


def kernel(inputs, W):
    xr = inputs.reshape(2, D, 4096)
    idx3 = _argmin_call(xr, W)
    q = _make_sc_gather()(W, idx3.reshape(-1))
    st, loss = _st_call(xr, q)
    out = st.reshape(2, D, 16, 16, 16)
    return (loss[0, 0], out)


# speedup vs baseline: 3.2792x; 1.0715x over previous
import functools

import jax
import jax.numpy as jnp
from jax import lax
from jax.experimental import pallas as pl
from jax.experimental.pallas import tpu as pltpu
from jax.experimental.pallas import tpu_sc as plsc

N_CODES = 8192
N_TOK = 8192
D = 256
C_COST = 0.25

BM = 1024
BN = 4096
BM3 = 2048


def _argmin_body(x_ref, w_ref, idx_ref, sw_ref):
    @pl.when(pl.program_id(0) == 0)
    def _():
        wv = w_ref[...]
        sw_ref[...] = jnp.sum(wv * wv, axis=1, keepdims=True)

    xt = x_ref[0]
    f2t = xt * 2.0
    sf = jnp.sum(xt * xt, axis=0, keepdims=True)
    s_iota = lax.broadcasted_iota(jnp.int32, (8, 1), 0).astype(jnp.float32)

    def chunk(t, carry):
        best_v, best_i = carry
        w = w_ref[pl.ds(t * BN, BN), :]
        mm2 = lax.dot_general(w, f2t, (((1,), (0,)), ((), ())),
                              preferred_element_type=jnp.float32)
        sw = sw_ref[pl.ds(t * BN, BN), :]
        n_g = BN // 8
        accs = []
        for k in range(4):
            acc_v = jnp.full((8, BM), jnp.inf, jnp.float32)
            acc_g = jnp.zeros((8, BM), jnp.float32)
            for g in range(k * n_g // 4, (k + 1) * n_g // 4):
                d_g = ((sf + sw[g * 8:(g + 1) * 8, :])
                       - mm2[g * 8:(g + 1) * 8, :])
                upd_g = d_g < acc_v
                acc_g = jnp.where(upd_g, jnp.float32(g), acc_g)
                acc_v = jnp.where(upd_g, d_g, acc_v)
            accs.append((acc_v, acc_g))
        run_v, run_g = accs[0]
        for acc_v, acc_g in accs[1:]:
            upd_k = acc_v < run_v
            run_g = jnp.where(upd_k, acc_g, run_g)
            run_v = jnp.where(upd_k, acc_v, run_v)
        vmin = jnp.min(run_v, axis=0, keepdims=True)
        glob = run_g * 8.0 + s_iota
        imin = jnp.min(jnp.where(run_v == vmin, glob, jnp.float32(jnp.inf)),
                       axis=0, keepdims=True)
        imin = imin + jnp.float32(BN) * t.astype(jnp.float32)
        upd = vmin < best_v
        return (jnp.where(upd, vmin, best_v),
                jnp.where(upd, imin, best_i))

    v0 = jnp.full((1, BM), jnp.inf, jnp.float32)
    i0 = jnp.zeros((1, BM), jnp.float32)
    _, best_i = lax.fori_loop(0, N_CODES // BN, chunk, (v0, i0),
                              unroll=2)
    idx_ref[...] = best_i.astype(jnp.int32).reshape(1, 1, BM)


_argmin_call = pl.pallas_call(
    _argmin_body,
    grid=(N_TOK // BM,),
    in_specs=[
        pl.BlockSpec((1, D, BM), lambda i: (i // (4096 // BM), 0,
                                            i % (4096 // BM))),
        pl.BlockSpec((N_CODES, D), lambda i: (0, 0)),
    ],
    out_specs=pl.BlockSpec((1, 1, BM), lambda i: (i, 0, 0)),
    out_shape=jax.ShapeDtypeStruct((N_TOK // BM, 1, BM), jnp.int32),
    scratch_shapes=[pltpu.VMEM((N_CODES, 1), jnp.float32)],
)


@functools.cache
def _make_sc_gather():
    info = plsc.get_sparse_core_info()
    nw = info.num_cores * info.num_subcores
    bpw = N_TOK // nw
    mesh = plsc.VectorSubcoreMesh(core_axis_name="c", subcore_axis_name="s")

    @functools.partial(
        pl.kernel, mesh=mesh,
        out_type=jax.ShapeDtypeStruct((N_TOK, D), jnp.float32),
        scratch_types=[
            pltpu.VMEM((bpw,), jnp.int32),
            pltpu.VMEM((bpw, D), jnp.float32),
            pltpu.SemaphoreType.DMA,
        ],
    )
    def gather(table_hbm, idx_hbm, out_hbm, idx_v, rows_v, sem):
        wid = lax.axis_index("s") * info.num_cores + lax.axis_index("c")
        base = wid * bpw
        pltpu.sync_copy(idx_hbm.at[pl.ds(base, bpw)], idx_v)
        pltpu.async_copy(table_hbm.at[idx_v], rows_v, sem).wait()
        pltpu.sync_copy(rows_v, out_hbm.at[pl.ds(base, bpw)])

    return gather


def _st_body(x_ref, q_ref, st_ref, loss_ref, acc_ref):
    i = pl.program_id(0)
    xt = x_ref[0]
    qt = jnp.transpose(q_ref[...], (1, 0))
    dif = qt - xt
    st_ref[...] = (xt + dif).reshape(1, D, BM3)
    part = jnp.sum(dif * dif)

    @pl.when(i == 0)
    def _():
        acc_ref[0] = part

    @pl.when(i > 0)
    def _():
        acc_ref[0] = acc_ref[0] + part

    @pl.when(i == pl.num_programs(0) - 1)
    def _():
        loss_ref[0, 0] = acc_ref[0] * ((1.0 + C_COST) / (N_TOK * D))


_st_call = pl.pallas_call(
    _st_body,
    grid=(N_TOK // BM3,),
    in_specs=[
        pl.BlockSpec((1, D, BM3), lambda i: (i // (4096 // BM3), 0,
                                             i % (4096 // BM3))),
        pl.BlockSpec((BM3, D), lambda i: (i, 0)),
    ],
    out_specs=[
        pl.BlockSpec((1, D, BM3), lambda i: (i // (4096 // BM3), 0,
                                             i % (4096 // BM3))),
        pl.BlockSpec(memory_space=pltpu.SMEM),
    ],
    out_shape=[
        jax.ShapeDtypeStruct((2, D, 4096), jnp.float32),
        jax.ShapeDtypeStruct((1, 1), jnp.float32),
    ],
    scratch_shapes=[pltpu.SMEM((1,), jnp.float32)],
)


def kernel(inputs, W):
    xr = inputs.reshape(2, D, 4096)
    idx3 = _argmin_call(xr, W)
    q = _make_sc_gather()(W, idx3.reshape(-1))
    st, loss = _st_call(xr, q)
    out = st.reshape(2, D, 16, 16, 16)
    return (loss[0, 0], out)


# speedup vs baseline: 3.3044x; 1.0077x over previous
import functools

import jax
import jax.numpy as jnp
from jax import lax
from jax.experimental import pallas as pl
from jax.experimental.pallas import tpu as pltpu
from jax.experimental.pallas import tpu_sc as plsc

N_CODES = 8192
N_TOK = 8192
D = 256
C_COST = 0.25

BM = 1024
BN = 4096
BM3 = 4096


def _argmin_body(x_ref, w_ref, idx_ref, sw_ref):
    @pl.when(pl.program_id(0) == 0)
    def _():
        wv = w_ref[...]
        sw_ref[...] = jnp.sum(wv * wv, axis=1, keepdims=True)

    xt = x_ref[0]
    f2t = xt * 2.0
    sf = jnp.sum(xt * xt, axis=0, keepdims=True)
    s_iota = lax.broadcasted_iota(jnp.int32, (8, 1), 0).astype(jnp.float32)

    def chunk(t, carry):
        best_v, best_i = carry
        w = w_ref[pl.ds(t * BN, BN), :]
        mm2 = lax.dot_general(w, f2t, (((1,), (0,)), ((), ())),
                              preferred_element_type=jnp.float32)
        sw = sw_ref[pl.ds(t * BN, BN), :]
        n_g = BN // 8
        accs = []
        for k in range(4):
            acc_v = jnp.full((8, BM), jnp.inf, jnp.float32)
            acc_g = jnp.zeros((8, BM), jnp.float32)
            for g in range(k * n_g // 4, (k + 1) * n_g // 4):
                d_g = ((sf + sw[g * 8:(g + 1) * 8, :])
                       - mm2[g * 8:(g + 1) * 8, :])
                upd_g = d_g < acc_v
                acc_g = jnp.where(upd_g, jnp.float32(g), acc_g)
                acc_v = jnp.where(upd_g, d_g, acc_v)
            accs.append((acc_v, acc_g))
        run_v, run_g = accs[0]
        for acc_v, acc_g in accs[1:]:
            upd_k = acc_v < run_v
            run_g = jnp.where(upd_k, acc_g, run_g)
            run_v = jnp.where(upd_k, acc_v, run_v)
        vmin = jnp.min(run_v, axis=0, keepdims=True)
        glob = run_g * 8.0 + s_iota
        imin = jnp.min(jnp.where(run_v == vmin, glob, jnp.float32(jnp.inf)),
                       axis=0, keepdims=True)
        imin = imin + jnp.float32(BN) * t.astype(jnp.float32)
        upd = vmin < best_v
        return (jnp.where(upd, vmin, best_v),
                jnp.where(upd, imin, best_i))

    v0 = jnp.full((1, BM), jnp.inf, jnp.float32)
    i0 = jnp.zeros((1, BM), jnp.float32)
    _, best_i = lax.fori_loop(0, N_CODES // BN, chunk, (v0, i0),
                              unroll=2)
    idx_ref[...] = best_i.astype(jnp.int32).reshape(1, 1, BM)


_argmin_call = pl.pallas_call(
    _argmin_body,
    grid=(N_TOK // BM,),
    in_specs=[
        pl.BlockSpec((1, D, BM), lambda i: (i // (4096 // BM), 0,
                                            i % (4096 // BM))),
        pl.BlockSpec((N_CODES, D), lambda i: (0, 0)),
    ],
    out_specs=pl.BlockSpec((1, 1, BM), lambda i: (i, 0, 0)),
    out_shape=jax.ShapeDtypeStruct((N_TOK // BM, 1, BM), jnp.int32),
    scratch_shapes=[pltpu.VMEM((N_CODES, 1), jnp.float32)],
)


@functools.cache
def _make_sc_gather():
    info = plsc.get_sparse_core_info()
    nw = info.num_cores * info.num_subcores
    bpw = N_TOK // nw
    mesh = plsc.VectorSubcoreMesh(core_axis_name="c", subcore_axis_name="s")

    @functools.partial(
        pl.kernel, mesh=mesh,
        out_type=jax.ShapeDtypeStruct((N_TOK, D), jnp.float32),
        scratch_types=[
            pltpu.VMEM((bpw,), jnp.int32),
            pltpu.VMEM((bpw, D), jnp.float32),
            pltpu.SemaphoreType.DMA,
        ],
    )
    def gather(table_hbm, idx_hbm, out_hbm, idx_v, rows_v, sem):
        wid = lax.axis_index("s") * info.num_cores + lax.axis_index("c")
        base = wid * bpw
        pltpu.sync_copy(idx_hbm.at[pl.ds(base, bpw)], idx_v)
        pltpu.async_copy(table_hbm.at[idx_v], rows_v, sem).wait()
        pltpu.sync_copy(rows_v, out_hbm.at[pl.ds(base, bpw)])

    return gather


def _st_body(x_ref, q_ref, st_ref, loss_ref, acc_ref):
    i = pl.program_id(0)
    xt = x_ref[0]
    qt = jnp.transpose(q_ref[...], (1, 0))
    dif = qt - xt
    st_ref[...] = (xt + dif).reshape(1, D, BM3)
    part = jnp.sum(dif * dif)

    @pl.when(i == 0)
    def _():
        acc_ref[0] = part

    @pl.when(i > 0)
    def _():
        acc_ref[0] = acc_ref[0] + part

    @pl.when(i == pl.num_programs(0) - 1)
    def _():
        loss_ref[0, 0] = acc_ref[0] * ((1.0 + C_COST) / (N_TOK * D))


_st_call = pl.pallas_call(
    _st_body,
    grid=(N_TOK // BM3,),
    in_specs=[
        pl.BlockSpec((1, D, BM3), lambda i: (i // (4096 // BM3), 0,
                                             i % (4096 // BM3))),
        pl.BlockSpec((BM3, D), lambda i: (i, 0)),
    ],
    out_specs=[
        pl.BlockSpec((1, D, BM3), lambda i: (i // (4096 // BM3), 0,
                                             i % (4096 // BM3))),
        pl.BlockSpec(memory_space=pltpu.SMEM),
    ],
    out_shape=[
        jax.ShapeDtypeStruct((2, D, 4096), jnp.float32),
        jax.ShapeDtypeStruct((1, 1), jnp.float32),
    ],
    scratch_shapes=[pltpu.SMEM((1,), jnp.float32)],
)


def kernel(inputs, W):
    xr = inputs.reshape(2, D, 4096)
    idx3 = _argmin_call(xr, W)
    q = _make_sc_gather()(W, idx3.reshape(-1))
    st, loss = _st_call(xr, q)
    out = st.reshape(2, D, 16, 16, 16)
    return (loss[0, 0], out)
